# baseline (speedup 1.0000x reference)
import functools

import jax
import jax.numpy as jnp
from jax.experimental import pallas as pl
from jax.experimental.pallas import tpu as pltpu

N_ROWS = 131072
N_TERMS = 64
NCORE = 2
BLOCK = 4096
NBLK = N_ROWS // (BLOCK * NCORE)
NBUF = 8


def _gj_body(k, carry):
    a, b = carry
    is_k_row = jax.lax.broadcasted_iota(jnp.int32, (N_TERMS, 1), 0) == k
    is_k_col = jax.lax.broadcasted_iota(jnp.int32, (1, N_TERMS), 1) == k
    row_k = jnp.sum(jnp.where(is_k_row, a, 0.0), axis=0, keepdims=True)
    pivot = jnp.sum(jnp.where(is_k_col, row_k, 0.0))
    inv_p = 1.0 / pivot
    norm_row = row_k * inv_p
    b_k = jnp.sum(jnp.where(is_k_row, b, 0.0)) * inv_p
    col = jnp.sum(jnp.where(is_k_col, a, 0.0), axis=1, keepdims=True)
    new_a = jnp.where(is_k_row, norm_row, a - col * norm_row)
    new_b = jnp.where(is_k_row, b_k, b - col * b_k)
    return new_a, new_b


def _solve_kernel(g_ref, r_ref, out_ref):
    gram = g_ref[0] + g_ref[1]
    rhs = r_ref[0] + r_ref[1]
    a, b = jax.lax.fori_loop(0, N_TERMS, _gj_body, (gram, rhs))
    out_ref[...] = b


def _make_partial():
    mesh = pltpu.create_tensorcore_mesh("c", num_cores=NCORE)

    def body(td_hbm, th_hbm, og_hbm, th_buf, td_buf, part_buf,
             sem_th, sem_td, sem_out):
        c = jax.lax.axis_index("c")
        base = c * (N_ROWS // NCORE)

        def th_copy(k):
            return pltpu.make_async_copy(
                th_hbm.at[pl.ds(base + k * BLOCK, BLOCK), :],
                th_buf.at[k % NBUF],
                sem_th.at[k % NBUF])

        def td_copy(k):
            return pltpu.make_async_copy(
                td_hbm.at[pl.ds(base + k * BLOCK, BLOCK), :],
                td_buf.at[k % NBUF],
                sem_td.at[k % NBUF])

        for k in range(NBUF):
            th_copy(k).start()
            td_copy(k).start()

        gram = jnp.zeros((N_TERMS, N_TERMS), jnp.float32)
        rhs = jnp.zeros((N_TERMS, 1), jnp.float32)
        for k in range(NBLK):
            th_copy(k).wait()
            td_copy(k).wait()
            th = th_buf[k % NBUF]
            td = td_buf[k % NBUF]
            if k + NBUF < NBLK:
                th_copy(k + NBUF).start()
                td_copy(k + NBUF).start()
            gram = gram + jax.lax.dot_general(
                th, th, (((0,), (0,)), ((), ())),
                preferred_element_type=jnp.float32,
                precision=jax.lax.Precision.DEFAULT)
            rhs = rhs + jax.lax.dot_general(
                th, td, (((0,), (0,)), ((), ())),
                preferred_element_type=jnp.float32,
                precision=jax.lax.Precision.DEFAULT)

        part_buf[0, :, :N_TERMS] = gram
        part_buf[0, :, N_TERMS:N_TERMS + 1] = rhs
        cp = pltpu.make_async_copy(
            part_buf, og_hbm.at[pl.ds(c, 1)], sem_out)
        cp.start()
        cp.wait()

    return pl.kernel(
        body,
        out_type=[
            jax.ShapeDtypeStruct((NCORE, N_TERMS, N_TERMS + 1), jnp.float32),
        ],
        mesh=mesh,
        scratch_types=[
            pltpu.VMEM((NBUF, BLOCK, N_TERMS), jnp.float32),
            pltpu.VMEM((NBUF, BLOCK, 1), jnp.float32),
            pltpu.VMEM((1, N_TERMS, N_TERMS + 1), jnp.float32),
            pltpu.SemaphoreType.DMA((NBUF,)),
            pltpu.SemaphoreType.DMA((NBUF,)),
            pltpu.SemaphoreType.DMA,
        ],
    )


_partial = _make_partial()


@functools.partial(jax.jit, static_argnames=())
def kernel(time_derivs, thetas):
    (pgr,) = _partial(time_derivs, thetas)
    pg = pgr[:, :, :N_TERMS]
    pr = pgr[:, :, N_TERMS:N_TERMS + 1]
    return pl.pallas_call(
        _solve_kernel,
        out_shape=jax.ShapeDtypeStruct((N_TERMS, 1), jnp.float32),
    )(pg, pr)


# speedup vs baseline: 1.0534x; 1.0534x over previous
import functools

import jax
import jax.numpy as jnp
from jax.experimental import pallas as pl
from jax.experimental.pallas import tpu as pltpu

N_ROWS = 131072
N_TERMS = 64
BLOCK_ROWS = 8192
GRID = N_ROWS // BLOCK_ROWS


def _gj_body(k, carry):
    a, b = carry
    is_k_row = jax.lax.broadcasted_iota(jnp.int32, (N_TERMS, 1), 0) == k
    is_k_col = jax.lax.broadcasted_iota(jnp.int32, (1, N_TERMS), 1) == k
    row_k = jnp.sum(jnp.where(is_k_row, a, 0.0), axis=0, keepdims=True)
    pivot = jnp.sum(jnp.where(is_k_col, row_k, 0.0))
    inv_p = 1.0 / pivot
    norm_row = row_k * inv_p
    b_k = jnp.sum(jnp.where(is_k_row, b, 0.0)) * inv_p
    col = jnp.sum(jnp.where(is_k_col, a, 0.0), axis=1, keepdims=True)
    new_a = jnp.where(is_k_row, norm_row, a - col * norm_row)
    new_b = jnp.where(is_k_row, b_k, b - col * b_k)
    return new_a, new_b


def _fit_kernel(td_ref, theta_ref, out_ref, gram_ref, rhs_ref):
    i = pl.program_id(0)
    th = theta_ref[...]
    part_g = jax.lax.dot_general(
        th, th, (((0,), (0,)), ((), ())),
        preferred_element_type=jnp.float32,
        precision=jax.lax.Precision.DEFAULT)
    part_r = jax.lax.dot_general(
        th, td_ref[...], (((0,), (0,)), ((), ())),
        preferred_element_type=jnp.float32,
        precision=jax.lax.Precision.DEFAULT)

    @pl.when(i == 0)
    def _():
        gram_ref[...] = part_g
        rhs_ref[...] = part_r

    @pl.when(i > 0)
    def _():
        gram_ref[...] += part_g
        rhs_ref[...] += part_r

    @pl.when(i == GRID - 1)
    def _():
        a, b = jax.lax.fori_loop(
            0, N_TERMS, _gj_body, (gram_ref[...], rhs_ref[...]))
        out_ref[...] = b


@functools.partial(jax.jit, static_argnames=())
def kernel(time_derivs, thetas):
    return pl.pallas_call(
        _fit_kernel,
        grid=(GRID,),
        in_specs=[
            pl.BlockSpec((BLOCK_ROWS, 1), lambda i: (i, 0)),
            pl.BlockSpec((BLOCK_ROWS, N_TERMS), lambda i: (i, 0)),
        ],
        out_specs=pl.BlockSpec((N_TERMS, 1), lambda i: (0, 0)),
        out_shape=jax.ShapeDtypeStruct((N_TERMS, 1), jnp.float32),
        scratch_shapes=[
            pltpu.VMEM((N_TERMS, N_TERMS), jnp.float32),
            pltpu.VMEM((N_TERMS, 1), jnp.float32),
        ],
    )(time_derivs, thetas)
